# baseline (speedup 1.0000x reference)
import functools

import jax
import jax.numpy as jnp
from jax import lax
from jax.experimental import pallas as pl
from jax.experimental.pallas import tpu as pltpu
from jax.experimental.pallas import tpu_sc as plsc

BATCH = 16384
EMBED = 32
NC = 2
NS = 16
NW = NC * NS
B_PER_W = BATCH // NW
NBUF = 16
ROUNDS = B_PER_W // 16


def _ring_control(sorted_idx):
    iota = jnp.arange(BATCH, dtype=jnp.int32)
    pan = sorted_idx >> 7
    kmod = iota & (B_PER_W - 1)
    w = iota >> 9
    prev = jnp.concatenate([pan[:1] - 1, pan[:-1]])
    fetch = (pan != prev) | (kmod == 0)
    m = (jnp.cumsum(fetch.reshape(NW, B_PER_W), axis=1)
         .reshape(-1).astype(jnp.int32) - 1)
    npan = m.reshape(NW, B_PER_W)[:, -1] + 1
    slot = m & (NBUF - 1)
    ps = jnp.zeros((BATCH,), jnp.int32).at[w * B_PER_W + m].set(pan << 7)
    nxt_fetch = jnp.concatenate([fetch[1:], jnp.ones((1,), fetch.dtype)])
    q2 = m + NBUF
    issue = nxt_fetch & (q2 < npan[w])
    parr = jnp.take(ps, jnp.clip(w * B_PER_W + q2, 0, BATCH - 1))
    k16 = jnp.arange(NW * NBUF, dtype=jnp.int32)
    pro_pos = (k16 >> 4) * B_PER_W + (k16 & (NBUF - 1))
    pro_start = (jnp.zeros((BATCH,), jnp.int32)
                 .at[pro_pos].set(jnp.take(ps, pro_pos)))
    pro_flag = (jnp.zeros((BATCH,), jnp.int32)
                .at[pro_pos].set(((k16 & (NBUF - 1))
                                  < npan[k16 >> 4]).astype(jnp.int32)))
    return (sorted_idx, fetch.astype(jnp.int32), slot,
            issue.astype(jnp.int32), parr, pro_start, pro_flag)


def _gather_sorted(tbl, ctl, out_ref, base, panels, stg, sems):
    sv, fv, slv, isv, pav, prs, prf = ctl
    rows0 = lax.iota(jnp.int32, 16)
    rows1 = rows0 + 16

    pstart = prs[pl.ds(0, 16)]
    pflag = prf[pl.ds(0, 16)]
    for k in range(NBUF):
        @pl.when(pflag[k] != 0)
        def _():
            st = pl.multiple_of(pstart[k], 128)
            pltpu.async_copy(tbl.at[:, pl.ds(st, 128)], panels.at[k],
                             sems.at[k])

    def round_body(g, carry):
        off = pl.multiple_of(g * 16, 8)
        v_s = sv[pl.ds(off, 16)]
        v_f = fv[pl.ds(off, 16)]
        v_sl = slv[pl.ds(off, 16)]
        v_is = isv[pl.ds(off, 16)]
        v_pa = pav[pl.ds(off, 16)]
        for k in range(16):
            sl = v_sl[k]

            @pl.when(v_f[k] != 0)
            def _():
                pltpu.make_async_copy(tbl.at[:, pl.ds(0, 128)],
                                      panels.at[sl], sems.at[sl]).wait()

            r = v_s[k]
            col = jnp.zeros((16,), jnp.int32) + (r & 127)
            v0 = plsc.load_gather(panels.at[sl], [rows0, col])
            v1 = plsc.load_gather(panels.at[sl], [rows1, col])
            row = (g * 16 + k) & 127
            stg[row, pl.ds(0, 16)] = v0
            stg[row, pl.ds(16, 16)] = v1

            @pl.when(v_is[k] != 0)
            def _():
                st = pl.multiple_of(v_pa[k], 128)
                pltpu.async_copy(tbl.at[:, pl.ds(st, 128)], panels.at[sl],
                                 sems.at[sl])

        @pl.when(lax.rem(g, 8) == 7)
        def _():
            off2 = pl.multiple_of(base + ((g // 8) << 7), 128)
            pltpu.sync_copy(stg, out_ref.at[pl.ds(off2, 128)])

        return carry

    lax.fori_loop(0, ROUNDS, round_body, 0)


def _sc_gather_body(*args):
    (uctl_hbm, ictl_hbm) = (args[0:7], args[7:14])
    (ugT, igT, umT, imT) = args[14:18]
    (out_ug, out_ig, out_um, out_im) = args[18:22]
    uctl = list(args[22:29])
    ictl = list(args[29:36])
    (panels, stg, sems) = args[36:39]
    c = lax.axis_index("c")
    s = lax.axis_index("s")
    wid = s * NC + c
    base = pl.multiple_of(wid * B_PER_W, 128)
    for a in range(7):
        pltpu.sync_copy(uctl_hbm[a].at[pl.ds(base, B_PER_W)], uctl[a])
        pltpu.sync_copy(ictl_hbm[a].at[pl.ds(base, B_PER_W)], ictl[a])
    _gather_sorted(ugT, uctl, out_ug, base, panels, stg, sems)
    _gather_sorted(umT, uctl, out_um, base, panels, stg, sems)
    _gather_sorted(igT, ictl, out_ig, base, panels, stg, sems)
    _gather_sorted(imT, ictl, out_im, base, panels, stg, sems)


@functools.cache
def _sc_gather():
  return pl.kernel(
    _sc_gather_body,
    out_type=[jax.ShapeDtypeStruct((BATCH, EMBED), jnp.float32)] * 4,
    mesh=plsc.VectorSubcoreMesh(core_axis_name="c", subcore_axis_name="s",
                                num_cores=NC, num_subcores=NS),
    scratch_types=[pltpu.VMEM((B_PER_W,), jnp.int32)] * 14 + [
        pltpu.VMEM((NBUF, EMBED, 128), jnp.float32),
        pltpu.VMEM((128, EMBED), jnp.float32),
        pltpu.SemaphoreType.DMA((NBUF,)),
    ],
    compiler_params=pltpu.CompilerParams(use_tc_tiling_on_sc=True,
                                         disable_bounds_checks=True,
                                         needs_layout_passes=False),
  )


CHUNK = 128
NCHUNK = B_PER_W // CHUNK


def _sc_unperm_body(inv_u_hbm, inv_i_hbm, sug, sig, sum_, sim,
                    out_ug, out_ig, out_um, out_im,
                    uidx, iidx, bug, big, bum, bim,
                    sem0, sem1, sem2, sem3):
    c = lax.axis_index("c")
    s = lax.axis_index("s")
    wid = s * NC + c
    base = pl.multiple_of(wid * B_PER_W, 128)
    pltpu.sync_copy(inv_u_hbm.at[wid], uidx)
    pltpu.sync_copy(inv_i_hbm.at[wid], iidx)
    cps = []
    for j in range(NCHUNK):
        dst = pl.ds(j * CHUNK, CHUNK)
        cps.append(pltpu.async_copy(sug.at[uidx.at[j]], bug.at[dst], sem0))
        cps.append(pltpu.async_copy(sig.at[iidx.at[j]], big.at[dst], sem1))
        cps.append(pltpu.async_copy(sum_.at[uidx.at[j]], bum.at[dst], sem2))
        cps.append(pltpu.async_copy(sim.at[iidx.at[j]], bim.at[dst], sem3))
    for cp in cps:
        cp.wait()
    out = pl.ds(base, B_PER_W)
    pltpu.sync_copy(bug, out_ug.at[out])
    pltpu.sync_copy(big, out_ig.at[out])
    pltpu.sync_copy(bum, out_um.at[out])
    pltpu.sync_copy(bim, out_im.at[out])


@functools.cache
def _sc_unperm():
  return pl.kernel(
    _sc_unperm_body,
    out_type=[jax.ShapeDtypeStruct((BATCH, EMBED), jnp.float32)] * 4,
    mesh=plsc.VectorSubcoreMesh(core_axis_name="c", subcore_axis_name="s",
                                num_cores=NC, num_subcores=NS),
    scratch_types=[
        pltpu.VMEM((NCHUNK, CHUNK), jnp.int32),
        pltpu.VMEM((NCHUNK, CHUNK), jnp.int32),
        pltpu.VMEM((B_PER_W, EMBED), jnp.float32),
        pltpu.VMEM((B_PER_W, EMBED), jnp.float32),
        pltpu.VMEM((B_PER_W, EMBED), jnp.float32),
        pltpu.VMEM((B_PER_W, EMBED), jnp.float32),
        pltpu.SemaphoreType.DMA,
        pltpu.SemaphoreType.DMA,
        pltpu.SemaphoreType.DMA,
        pltpu.SemaphoreType.DMA,
    ],
    compiler_params=pltpu.CompilerParams(use_tc_tiling_on_sc=False),
  )


def _tc_mlp_body(ug, ig, um, im, w1a, w1b, b1, w2, b2, w3, b3, w4, b4,
                 wfg, wfh, bf, out_ref):
    dot = functools.partial(jnp.dot, preferred_element_type=jnp.float32)
    gmf = ug[...] * ig[...]
    h = jnp.maximum(dot(um[...], w1a[...]) + dot(im[...], w1b[...]) + b1[...], 0.0)
    h = jnp.maximum(dot(h, w2[...]) + b2[...], 0.0)
    h = jnp.maximum(dot(h, w3[...]) + b3[...], 0.0)
    h = jnp.maximum(dot(h, w4[...]) + b4[...], 0.0)
    logit = (jnp.sum(gmf * wfg[...], axis=1) + jnp.sum(h * wfh[...], axis=1)
             + bf[0])
    out_ref[...] = 1.0 / (1.0 + jnp.exp(-logit))


TCB = 4096


def kernel(user, item, ue_gmf, ie_gmf, ue_mlp, ie_mlp,
           W1, b1, W2, b2, W3, b3, W4, b4, Wf, bf):
    user = user.astype(jnp.int32)
    item = item.astype(jnp.int32)
    iota = jnp.arange(BATCH, dtype=jnp.int32)
    us, pu = lax.sort((user, iota), num_keys=1)
    its, pi_ = lax.sort((item, iota), num_keys=1)
    inv_u = jnp.zeros((BATCH,), jnp.int32).at[pu].set(iota)
    inv_i = jnp.zeros((BATCH,), jnp.int32).at[pi_].set(iota)

    uctl = _ring_control(us)
    ictl = _ring_control(its)
    sug, sig, sum_, sim = _sc_gather()(
        *uctl, *ictl, ue_gmf.T, ie_gmf.T, ue_mlp.T, ie_mlp.T)
    ug, ig, um, im = _sc_unperm()(
        inv_u.reshape(NW, NCHUNK, CHUNK), inv_i.reshape(NW, NCHUNK, CHUNK),
        sug, sig, sum_, sim)

    weights = (W1[:EMBED], W1[EMBED:], b1.reshape(1, -1),
               W2, b2.reshape(1, -1), W3, b3.reshape(1, -1),
               W4, b4.reshape(1, -1),
               Wf[:EMBED].T, Wf[EMBED:].T)
    row_spec = pl.BlockSpec((TCB, EMBED), lambda i: (i, 0))
    wspecs = [pl.BlockSpec(w.shape, lambda i: (0, 0)) for w in weights]
    out = pl.pallas_call(
        _tc_mlp_body,
        grid=(BATCH // TCB,),
        in_specs=[row_spec] * 4 + wspecs
        + [pl.BlockSpec(memory_space=pltpu.SMEM)],
        out_specs=pl.BlockSpec((TCB,), lambda i: (i,)),
        out_shape=jax.ShapeDtypeStruct((BATCH,), jnp.float32),
    )(ug, ig, um, im, *weights, bf)
    return out


# speedup vs baseline: 1.2677x; 1.2677x over previous
import functools

import jax
import jax.numpy as jnp
from jax import lax
from jax.experimental import pallas as pl
from jax.experimental.pallas import tpu as pltpu
from jax.experimental.pallas import tpu_sc as plsc

BATCH = 16384
EMBED = 32
NC = 2
NS = 16
NW = NC * NS
B_PER_W = BATCH // NW
NBUF = 16
ROUNDS = B_PER_W // NBUF
FLUSH = 128 // NBUF


def _gather_one_table(tbl, idx_v, out_ref, base, panels, stg, sems):
    rows0 = lax.iota(jnp.int32, 16)
    rows1 = rows0 + 16

    def issue(r, slot):
        start = pl.multiple_of((r >> 7) << 7, 128)
        pltpu.async_copy(tbl.at[:, pl.ds(start, 128)], panels.at[slot],
                         sems[slot])

    v_first = idx_v[pl.ds(0, 16)]
    for k in range(NBUF):
        issue(v_first[k], k)

    def round_body(g, v_cur):
        nxt = pl.multiple_of(lax.min((g + 1) * NBUF, B_PER_W - NBUF), 8)
        v_next = idx_v[pl.ds(nxt, 16)]
        for k in range(NBUF):
            pltpu.make_async_copy(tbl.at[:, pl.ds(0, 128)], panels.at[k],
                                  sems[k]).wait()
            r = v_cur[k]
            col = jnp.zeros((16,), jnp.int32) + (r & 127)
            v0 = plsc.load_gather(panels.at[k], [rows0, col])
            v1 = plsc.load_gather(panels.at[k], [rows1, col])
            j = jnp.zeros((16,), jnp.int32) + ((g * NBUF + k) & 127)
            plsc.store_scatter(stg, [rows0, j], v0)
            plsc.store_scatter(stg, [rows1, j], v1)

            @pl.when(g < ROUNDS - 1)
            def _():
                issue(v_next[k], k)

        @pl.when(lax.rem(g, FLUSH) == FLUSH - 1)
        def _():
            off = pl.multiple_of(base + ((g // FLUSH) << 7), 128)
            pltpu.sync_copy(stg, out_ref.at[:, pl.ds(off, 128)])

        return v_next

    lax.fori_loop(0, ROUNDS, round_body, v_first)


def _sc_gather_body(user_hbm, item_hbm, ugT, igT, umT, imT,
                    out_ug, out_ig, out_um, out_im,
                    uidx_v, iidx_v, panels, stg, *sems):
    c = lax.axis_index("c")
    s = lax.axis_index("s")
    wid = s * NC + c
    base = pl.multiple_of(wid * B_PER_W, 128)
    pltpu.sync_copy(user_hbm.at[pl.ds(base, B_PER_W)], uidx_v)
    pltpu.sync_copy(item_hbm.at[pl.ds(base, B_PER_W)], iidx_v)
    _gather_one_table(ugT, uidx_v, out_ug, base, panels, stg, sems)
    _gather_one_table(igT, iidx_v, out_ig, base, panels, stg, sems)
    _gather_one_table(umT, uidx_v, out_um, base, panels, stg, sems)
    _gather_one_table(imT, iidx_v, out_im, base, panels, stg, sems)


@functools.cache
def _sc_gather():
  return pl.kernel(
    _sc_gather_body,
    out_type=[jax.ShapeDtypeStruct((EMBED, BATCH), jnp.float32)] * 4,
    mesh=plsc.VectorSubcoreMesh(core_axis_name="c", subcore_axis_name="s",
                                num_cores=NC, num_subcores=NS),
    scratch_types=[
        pltpu.VMEM((B_PER_W,), jnp.int32),
        pltpu.VMEM((B_PER_W,), jnp.int32),
        pltpu.VMEM((NBUF, EMBED, 128), jnp.float32),
        pltpu.VMEM((EMBED, 128), jnp.float32),
    ] + [pltpu.SemaphoreType.DMA] * NBUF,
    compiler_params=pltpu.CompilerParams(use_tc_tiling_on_sc=True,
                                         disable_bounds_checks=True,
                                         needs_layout_passes=False),
  )


def _tc_mlp_body(ug, ig, um, im, w1a, w1b, b1, w2, b2, w3, b3, w4, b4,
                 wfg, wfh, bf, out_ref):
    dot = functools.partial(jnp.dot, preferred_element_type=jnp.float32)
    gmf = ug[...] * ig[...]
    h = jnp.maximum(dot(w1a[...], um[...]) + dot(w1b[...], im[...]) + b1[...], 0.0)
    h = jnp.maximum(dot(w2[...], h) + b2[...], 0.0)
    h = jnp.maximum(dot(w3[...], h) + b3[...], 0.0)
    h = jnp.maximum(dot(w4[...], h) + b4[...], 0.0)
    logit = dot(wfg[...], gmf) + dot(wfh[...], h) + bf[0]
    out_ref[...] = 1.0 / (1.0 + jnp.exp(-logit))


TCB = 4096


def kernel(user, item, ue_gmf, ie_gmf, ue_mlp, ie_mlp,
           W1, b1, W2, b2, W3, b3, W4, b4, Wf, bf):
    user = user.astype(jnp.int32)
    item = item.astype(jnp.int32)
    ug_o, ig_o, um_o, im_o = _sc_gather()(
        user, item, ue_gmf.T, ie_gmf.T, ue_mlp.T, ie_mlp.T)

    weights = (W1[:EMBED].T, W1[EMBED:].T, b1.reshape(-1, 1),
               W2.T, b2.reshape(-1, 1), W3.T, b3.reshape(-1, 1),
               W4.T, b4.reshape(-1, 1),
               Wf[:EMBED].reshape(1, -1), Wf[EMBED:].reshape(1, -1))
    col_spec = pl.BlockSpec((EMBED, TCB), lambda i: (0, i))
    wspecs = [pl.BlockSpec(w.shape, lambda i: (0, 0)) for w in weights]
    out = pl.pallas_call(
        _tc_mlp_body,
        grid=(BATCH // TCB,),
        in_specs=[col_spec] * 4 + wspecs
        + [pl.BlockSpec(memory_space=pltpu.SMEM)],
        out_specs=pl.BlockSpec((1, TCB), lambda i: (0, i)),
        out_shape=jax.ShapeDtypeStruct((1, BATCH), jnp.float32),
    )(ug_o, ig_o, um_o, im_o, *weights, bf)
    return out.reshape(BATCH)
